# Initial kernel scaffold; baseline (speedup 1.0000x reference)
import jax
import jax.numpy as jnp
from jax.experimental import pallas as pl


def kernel(x, Wg, bg, W1, b1, W2, b2):
    raise NotImplementedError("write your pallas kernel here")


# speedup vs baseline: 2.2667x; 2.2667x over previous
import jax
import jax.numpy as jnp
from jax.experimental import pallas as pl
from jax.experimental.pallas import tpu as pltpu

BT = 256

_INV_SQRT2 = 0.7071067811865476


def _gelu_exact(h):
    return 0.5 * h * (1.0 + jax.lax.erf(h * _INV_SQRT2))


def _gate_body(x_ref, wg_ref, bg_ref, gate_ref):
    logits = jnp.dot(x_ref[...], wg_ref[...],
                     preferred_element_type=jnp.float32) + bg_ref[0, :][None, :]
    m = jnp.max(logits, axis=1, keepdims=True)
    p = jnp.exp(logits - m)
    p = p / jnp.sum(p, axis=1, keepdims=True)
    T, E = p.shape
    eidx = jax.lax.broadcasted_iota(jnp.int32, (T, E), 1)
    m0 = jnp.max(p, axis=1, keepdims=True)
    i0 = jnp.min(jnp.where(p == m0, eidx, E), axis=1, keepdims=True)
    oh0 = eidx == i0
    p1 = jnp.where(oh0, -1.0, p)
    m1 = jnp.max(p1, axis=1, keepdims=True)
    i1 = jnp.min(jnp.where(p1 == m1, eidx, E), axis=1, keepdims=True)
    oh1 = eidx == i1
    gate_ref[...] = jnp.where(oh0 | oh1, p, 0.0)


def _ffn_dense_body(x_ref, gate_ref, w1_ref, b1_ref, w2_ref, b2_ref, y_ref):
    e = pl.program_id(0)
    f = pl.program_id(1)
    t = pl.program_id(2)
    tsl = pl.ds(t * BT, BT)
    xb = x_ref[tsl, :]
    h = jnp.dot(xb, w1_ref[0],
                preferred_element_type=jnp.float32) + b1_ref[0, 0, :][None, :]
    h = _gelu_exact(h)
    o = jnp.dot(h, w2_ref[0],
                preferred_element_type=jnp.float32)
    o = jnp.where(f == 0, o + b2_ref[0, 0, :][None, :], o)
    gb = gate_ref[...]
    eidx = jax.lax.broadcasted_iota(jnp.int32, gb.shape, 1)
    g = jnp.sum(jnp.where(eidx == e, gb, 0.0), axis=1)
    contrib = o * g[:, None]

    @pl.when((e == 0) & (f == 0))
    def _():
        y_ref[tsl, :] = contrib

    @pl.when((e > 0) | (f > 0))
    def _():
        y_ref[tsl, :] = y_ref[tsl, :] + contrib


def kernel(x, Wg, bg, W1, b1, W2, b2):
    B, S, D = x.shape
    T = B * S
    E = Wg.shape[1]
    FF = W1.shape[2]
    xf = x.reshape(T, D)

    gate = pl.pallas_call(
        _gate_body,
        out_shape=jax.ShapeDtypeStruct((T, E), jnp.float32),
    )(xf, Wg, bg.reshape(1, E))

    FFB = FF // 2
    y = pl.pallas_call(
        _ffn_dense_body,
        grid=(E, FF // FFB, T // BT),
        in_specs=[
            pl.BlockSpec((T, D), lambda e, f, t: (0, 0)),
            pl.BlockSpec((BT, E), lambda e, f, t: (t, 0)),
            pl.BlockSpec((1, D, FFB), lambda e, f, t: (e, 0, f)),
            pl.BlockSpec((1, 1, FFB), lambda e, f, t: (e, 0, f)),
            pl.BlockSpec((1, FFB, D), lambda e, f, t: (e, f, 0)),
            pl.BlockSpec((1, 1, D), lambda e, f, t: (e, 0, 0)),
        ],
        out_specs=pl.BlockSpec((T, D), lambda e, f, t: (0, 0)),
        out_shape=jax.ShapeDtypeStruct((T, D), jnp.float32),
    )(xf, gate, W1, b1.reshape(E, 1, FF), W2, b2.reshape(E, 1, D))
    return y.reshape(B, S, D)
